# baseline (speedup 1.0000x reference)
import functools

import numpy as np
import jax
import jax.numpy as jnp
from jax import lax
from jax.experimental import pallas as pl
from jax.experimental.pallas import tpu as pltpu
from jax.experimental.pallas import tpu_sc as plsc

_LENS = (4096, 3584, 3072, 2560, 2048, 1536, 1024, 512)
_D = 256
_TOTAL = sum(_LENS)
_NC = 2
_NS = 16
_NW = _NC * _NS


def _metadata():
    lengths = np.array(_LENS, np.int64)
    max_len = int(lengths.max())
    bs = (lengths[None, :] > np.arange(max_len)[:, None]).sum(axis=1)
    off = np.zeros(max_len, np.int64)
    off[1:] = np.cumsum(bs)[:-1]
    dests = [(off[:L] + j).astype(np.int32) for j, L in enumerate(_LENS)]
    sorted_idx = np.argsort(-lengths, kind="stable")
    unsorted_idx = np.argsort(sorted_idx)
    return bs, sorted_idx, unsorted_idx, dests


_BS, _SORTED, _UNSORTED, _DESTS = _metadata()
_CNTS = tuple(L // _NW for L in _LENS)


_NSLOT = 5
_SLOT_ROWS = tuple(
    max(_CNTS[b::_NSLOT]) for b in range(_NSLOT)
)


def _pack_body(*refs):
    xs = refs[0:8]
    out = refs[8]
    bufs = refs[9 : 9 + _NSLOT]
    idxs = refs[9 + _NSLOT : 17 + _NSLOT]
    gsem = refs[17 + _NSLOT : 17 + 2 * _NSLOT]
    ssem = refs[17 + 2 * _NSLOT : 17 + 3 * _NSLOT]
    wid = lax.axis_index("s") * _NC + lax.axis_index("c")

    def gather(j):
        cnt = _CNTS[j]
        return pltpu.async_copy(
            xs[j].at[pl.ds(wid * cnt, cnt)],
            bufs[j % _NSLOT].at[pl.ds(0, cnt)],
            gsem[j % _NSLOT],
        )

    gh = [gather(0), gather(1)] + [None] * (_NSLOT - 2)
    sh = [None] * _NSLOT
    lane = lax.iota(jnp.int32, 16)
    for j in range(8):
        cnt = _CNTS[j]
        base = wid * cnt
        for c in range(cnt // 16):
            t = base + (c * 16 + lane)
            k = lax.shift_right_logical(t, 9)
            r = lax.bitwise_and(t, 511)
            dest = 256 * k * (17 - k) + r * (8 - k) + j
            idxs[j][pl.ds(c * 16, 16)] = dest
    for j in range(8):
        b = j % _NSLOT
        cnt = _CNTS[j]
        gh[b].wait()
        sh[b] = pltpu.async_copy(
            bufs[b].at[pl.ds(0, cnt)], out.at[idxs[j]], ssem[b]
        )
        if j + 2 < 8:
            if j >= 3:
                sh[(j - 3) % _NSLOT].wait()
            gh[(j + 2) % _NSLOT] = gather(j + 2)
    for j in range(3, 8):
        sh[j % _NSLOT].wait()


_pack = functools.partial(
    pl.kernel,
    mesh=plsc.VectorSubcoreMesh(core_axis_name="c", subcore_axis_name="s"),
    out_type=jax.ShapeDtypeStruct((_TOTAL, _D), jnp.float32),
    scratch_types=[pltpu.VMEM((r, _D), jnp.float32) for r in _SLOT_ROWS]
    + [pltpu.VMEM((c,), jnp.int32) for c in _CNTS]
    + [pltpu.SemaphoreType.DMA for _ in range(_NSLOT)]
    + [pltpu.SemaphoreType.DMA for _ in range(_NSLOT)],
)(_pack_body)


def kernel(x0, x1, x2, x3, x4, x5, x6, x7):
    xs = (x0, x1, x2, x3, x4, x5, x6, x7)
    data = _pack(*xs)
    return (
        data,
        jnp.asarray(_BS, dtype=jnp.int64),
        jnp.asarray(_SORTED, dtype=jnp.int64),
        jnp.asarray(_UNSORTED, dtype=jnp.int64),
    )


# speedup vs baseline: 1.0044x; 1.0044x over previous
import functools

import numpy as np
import jax
import jax.numpy as jnp
from jax import lax
from jax.experimental import pallas as pl
from jax.experimental.pallas import tpu as pltpu
from jax.experimental.pallas import tpu_sc as plsc

_LENS = (4096, 3584, 3072, 2560, 2048, 1536, 1024, 512)
_D = 256
_TOTAL = sum(_LENS)
_NC = 2
_NS = 16
_NW = _NC * _NS


def _metadata():
    lengths = np.array(_LENS, np.int64)
    max_len = int(lengths.max())
    bs = (lengths[None, :] > np.arange(max_len)[:, None]).sum(axis=1)
    off = np.zeros(max_len, np.int64)
    off[1:] = np.cumsum(bs)[:-1]
    dests = [(off[:L] + j).astype(np.int32) for j, L in enumerate(_LENS)]
    sorted_idx = np.argsort(-lengths, kind="stable")
    unsorted_idx = np.argsort(sorted_idx)
    return bs, sorted_idx, unsorted_idx, dests


_BS, _SORTED, _UNSORTED, _DESTS = _metadata()
_CNTS = tuple(L // _NW for L in _LENS)


_NSLOT = 5
_SLOT_ROWS = tuple(
    max(_CNTS[b::_NSLOT]) for b in range(_NSLOT)
)


def _pack_body(*refs):
    xs = refs[0:8]
    out = refs[8]
    bufs = refs[9 : 9 + _NSLOT]
    idxs = refs[9 + _NSLOT : 17 + _NSLOT]
    gsem = refs[17 + _NSLOT : 17 + 2 * _NSLOT]
    ssem = refs[17 + 2 * _NSLOT : 17 + 3 * _NSLOT]
    wid = lax.axis_index("s") * _NC + lax.axis_index("c")

    def gather(j):
        cnt = _CNTS[j]
        return pltpu.async_copy(
            xs[j].at[pl.ds(wid * cnt, cnt)],
            bufs[j % _NSLOT].at[pl.ds(0, cnt)],
            gsem[j % _NSLOT],
        )

    gh = [gather(0), gather(1)] + [None] * (_NSLOT - 2)
    sh = [None] * _NSLOT
    lane = lax.iota(jnp.int32, 16)
    for j in range(8):
        cnt = _CNTS[j]
        base = wid * cnt

        def chunk(c, carry, j=j, base=base):
            t = base + (c * 16 + lane)
            k = lax.shift_right_logical(t, 9)
            r = lax.bitwise_and(t, 511)
            dest = 256 * k * (17 - k) + r * (8 - k) + j
            idxs[j][pl.ds(c * 16, 16)] = dest
            return carry

        lax.fori_loop(0, cnt // 16, chunk, 0, unroll=False)
    for j in range(8):
        b = j % _NSLOT
        cnt = _CNTS[j]
        gh[b].wait()
        sh[b] = pltpu.async_copy(
            bufs[b].at[pl.ds(0, cnt)], out.at[idxs[j]], ssem[b]
        )
        if j + 2 < 8:
            if j >= 3:
                sh[(j - 3) % _NSLOT].wait()
            gh[(j + 2) % _NSLOT] = gather(j + 2)
    for j in range(3, 8):
        sh[j % _NSLOT].wait()


_pack = functools.partial(
    pl.kernel,
    mesh=plsc.VectorSubcoreMesh(core_axis_name="c", subcore_axis_name="s"),
    out_type=jax.ShapeDtypeStruct((_TOTAL, _D), jnp.float32),
    scratch_types=[pltpu.VMEM((r, _D), jnp.float32) for r in _SLOT_ROWS]
    + [pltpu.VMEM((c,), jnp.int32) for c in _CNTS]
    + [pltpu.SemaphoreType.DMA for _ in range(_NSLOT)]
    + [pltpu.SemaphoreType.DMA for _ in range(_NSLOT)],
)(_pack_body)


def kernel(x0, x1, x2, x3, x4, x5, x6, x7):
    xs = (x0, x1, x2, x3, x4, x5, x6, x7)
    data = _pack(*xs)
    return (
        data,
        jnp.asarray(_BS, dtype=jnp.int64),
        jnp.asarray(_SORTED, dtype=jnp.int64),
        jnp.asarray(_UNSORTED, dtype=jnp.int64),
    )


# speedup vs baseline: 1.0522x; 1.0476x over previous
import functools

import numpy as np
import jax
import jax.numpy as jnp
from jax import lax
from jax.experimental import pallas as pl
from jax.experimental.pallas import tpu as pltpu
from jax.experimental.pallas import tpu_sc as plsc

_LENS = (4096, 3584, 3072, 2560, 2048, 1536, 1024, 512)
_D = 256
_TOTAL = sum(_LENS)
_T = max(_LENS)
_NC = 2
_NS = 16
_NW = _NC * _NS
_CNTS = tuple(L // _NW for L in _LENS)
_NSLOT = 5
_SLOT_ROWS = tuple(max(_CNTS[b::_NSLOT]) for b in range(_NSLOT))
_BS_PER_W = _T // _NW


def _pack_body(*refs):
    xs = refs[0:8]
    out = refs[8]
    bs_out = refs[9]
    sort_out = refs[10]
    unsort_out = refs[11]
    bufs = refs[12 : 12 + _NSLOT]
    idxs = refs[12 + _NSLOT : 20 + _NSLOT]
    bs_v = refs[20 + _NSLOT]
    perm_v = refs[21 + _NSLOT]
    gsem = refs[22 + _NSLOT : 22 + 2 * _NSLOT]
    ssem = refs[22 + 2 * _NSLOT : 22 + 3 * _NSLOT]
    csem = refs[22 + 3 * _NSLOT]
    wid = lax.axis_index("s") * _NC + lax.axis_index("c")

    def gather(j):
        cnt = _CNTS[j]
        return pltpu.async_copy(
            xs[j].at[pl.ds(wid * cnt, cnt)],
            bufs[j % _NSLOT].at[pl.ds(0, cnt)],
            gsem[j % _NSLOT],
        )

    gh = [gather(0), gather(1)] + [None] * (_NSLOT - 2)
    sh = [None] * _NSLOT
    lane = lax.iota(jnp.int32, 16)

    bs_base = wid * _BS_PER_W

    def bs_chunk(c, carry):
        t = bs_base + (c * 16 + lane)
        bs_v[pl.ds(c * 16, 16)] = 8 - lax.shift_right_logical(t, 9)
        return carry

    lax.fori_loop(0, _BS_PER_W // 16, bs_chunk, 0, unroll=False)
    ch = [pltpu.async_copy(bs_v, bs_out.at[pl.ds(bs_base, _BS_PER_W)], csem)]
    perm_v[...] = lane

    @pl.when(wid == 0)
    def _():
        pltpu.sync_copy(perm_v.at[pl.ds(0, 8)], sort_out)
        pltpu.sync_copy(perm_v.at[pl.ds(0, 8)], unsort_out)

    for j in range(8):
        cnt = _CNTS[j]
        base = wid * cnt

        def chunk(c, carry, j=j, base=base):
            t = base + (c * 16 + lane)
            k = lax.shift_right_logical(t, 9)
            r = lax.bitwise_and(t, 511)
            dest = 256 * k * (17 - k) + r * (8 - k) + j
            idxs[j][pl.ds(c * 16, 16)] = dest
            return carry

        lax.fori_loop(0, cnt // 16, chunk, 0, unroll=False)

    for j in range(8):
        b = j % _NSLOT
        cnt = _CNTS[j]
        gh[b].wait()
        sh[b] = pltpu.async_copy(
            bufs[b].at[pl.ds(0, cnt)], out.at[idxs[j]], ssem[b]
        )
        if j + 2 < 8:
            if j >= 3:
                sh[(j - 3) % _NSLOT].wait()
            gh[(j + 2) % _NSLOT] = gather(j + 2)
    for j in range(3, 8):
        sh[j % _NSLOT].wait()
    for h in ch:
        h.wait()


_pack = functools.partial(
    pl.kernel,
    mesh=plsc.VectorSubcoreMesh(core_axis_name="c", subcore_axis_name="s"),
    out_type=(
        jax.ShapeDtypeStruct((_TOTAL, _D), jnp.float32),
        jax.ShapeDtypeStruct((_T,), jnp.int32),
        jax.ShapeDtypeStruct((8,), jnp.int32),
        jax.ShapeDtypeStruct((8,), jnp.int32),
    ),
    scratch_types=[pltpu.VMEM((r, _D), jnp.float32) for r in _SLOT_ROWS]
    + [pltpu.VMEM((c,), jnp.int32) for c in _CNTS]
    + [pltpu.VMEM((_BS_PER_W,), jnp.int32)]
    + [pltpu.VMEM((16,), jnp.int32)]
    + [pltpu.SemaphoreType.DMA for _ in range(_NSLOT)]
    + [pltpu.SemaphoreType.DMA for _ in range(_NSLOT)]
    + [pltpu.SemaphoreType.DMA],
)(_pack_body)


def kernel(x0, x1, x2, x3, x4, x5, x6, x7):
    xs = (x0, x1, x2, x3, x4, x5, x6, x7)
    data, bs, sort_idx, unsort_idx = _pack(*xs)
    return (
        data,
        bs.astype(jnp.int64),
        sort_idx.astype(jnp.int64),
        unsort_idx.astype(jnp.int64),
    )
